# baseline (speedup 1.0000x reference)
import jax
import jax.numpy as jnp
from jax import lax
from jax.experimental import pallas as pl
from jax.experimental.pallas import tpu as pltpu
from jax.experimental.pallas import tpu_sc as plsc

B, L, V, D = 4096, 200, 100000, 128
LANES = 16
NW = 32
N_PER_W = (B * L) // NW
C = 32
G = N_PER_W // C
NBUF = 8


def _body(tok_hbm, idx_hbm, pos_hbm, out_hbm,
          idx_v, pos_v, rows, obuf, semg, sems):
    wid = lax.axis_index("s") * 2 + lax.axis_index("c")
    wbase = wid * N_PER_W

    pltpu.sync_copy(idx_hbm.at[pl.ds(wbase, N_PER_W)], idx_v)

    def start_gather(g, b):
        pltpu.async_copy(tok_hbm.at[idx_v.at[pl.ds(g * C, C)]], rows[b], semg[b])

    def gather_wait(b):
        pltpu.make_async_copy(tok_hbm.at[pl.ds(0, C)], rows[b], semg[b]).wait()

    def scatter_wait(b):
        pltpu.make_async_copy(obuf[b], out_hbm.at[pl.ds(0, C)], sems[b]).wait()

    for b in range(NBUF):
        start_gather(b, b)
    pltpu.sync_copy(pos_hbm, pos_v)

    def outer(go, _):
        for b in range(NBUF):
            g = go * NBUF + b
            gather_wait(b)

            @pl.when(go > 0)
            def _():
                scatter_wait(b)

            @plsc.parallel_loop(0, C, 1, unroll=2)
            def row(r):
                lr = lax.rem(g * C + r, L)
                for c in range(D // LANES):
                    sl = pl.ds(c * LANES, LANES)
                    obuf[b][r, sl] = rows[b][r, sl] + pos_v[lr, sl]

            @pl.when(g + NBUF < G)
            def _():
                start_gather(g + NBUF, b)

            pltpu.async_copy(obuf[b], out_hbm.at[pl.ds(wbase + g * C, C)], sems[b])
        return 0

    lax.fori_loop(0, G // NBUF, outer, 0)
    for b in range(NBUF):
        scatter_wait(b)


@jax.jit
def _embed(idx_flat, token_table, pos_table):
    mesh = plsc.VectorSubcoreMesh(core_axis_name="c", subcore_axis_name="s")
    k = pl.kernel(
        _body,
        out_type=jax.ShapeDtypeStruct((B * L, D), jnp.float32),
        mesh=mesh,
        scratch_types=[
            pltpu.VMEM((N_PER_W,), jnp.int32),
            pltpu.VMEM((L, D), jnp.float32),
            [pltpu.VMEM((C, D), jnp.float32) for _ in range(NBUF)],
            [pltpu.VMEM((C, D), jnp.float32) for _ in range(NBUF)],
            [pltpu.SemaphoreType.DMA for _ in range(NBUF)],
            [pltpu.SemaphoreType.DMA for _ in range(NBUF)],
        ],
    )
    return k(token_table, idx_flat, pos_table)


def kernel(inputs, token_table, pos_table):
    idx_flat = inputs.reshape(B * L).astype(jnp.int32)
    out = _embed(idx_flat, token_table, pos_table)
    return out.reshape(B, L, D)


# speedup vs baseline: 1.0118x; 1.0118x over previous
import jax
import jax.numpy as jnp
from jax import lax
from jax.experimental import pallas as pl
from jax.experimental.pallas import tpu as pltpu
from jax.experimental.pallas import tpu_sc as plsc

B, L, V, D = 4096, 200, 100000, 128
LANES = 16
NW = 32
N_PER_W = (B * L) // NW
C = 64
G = N_PER_W // C
NBUF = 4


def _body(tok_hbm, idx_hbm, pos_hbm, out_hbm,
          idx_v, pos_v, rows, obuf, semg, sems):
    wid = lax.axis_index("s") * 2 + lax.axis_index("c")
    wbase = wid * N_PER_W

    pltpu.sync_copy(idx_hbm.at[pl.ds(wbase, N_PER_W)], idx_v)

    def start_gather(g, b):
        pltpu.async_copy(tok_hbm.at[idx_v.at[pl.ds(g * C, C)]], rows[b], semg[b])

    def gather_wait(b):
        pltpu.make_async_copy(tok_hbm.at[pl.ds(0, C)], rows[b], semg[b]).wait()

    def scatter_wait(b):
        pltpu.make_async_copy(obuf[b], out_hbm.at[pl.ds(0, C)], sems[b]).wait()

    for b in range(NBUF):
        start_gather(b, b)
    pltpu.sync_copy(pos_hbm, pos_v)

    def outer(go, _):
        for b in range(NBUF):
            g = go * NBUF + b
            gather_wait(b)

            @pl.when(go > 0)
            def _():
                scatter_wait(b)

            @plsc.parallel_loop(0, C, 1, unroll=4)
            def row(r):
                lr = lax.rem(g * C + r, L)
                for c in range(D // LANES):
                    sl = pl.ds(c * LANES, LANES)
                    obuf[b][r, sl] = rows[b][r, sl] + pos_v[lr, sl]

            @pl.when(g + NBUF < G)
            def _():
                start_gather(g + NBUF, b)

            pltpu.async_copy(obuf[b], out_hbm.at[pl.ds(wbase + g * C, C)], sems[b])
        return 0

    lax.fori_loop(0, G // NBUF, outer, 0)
    for b in range(NBUF):
        scatter_wait(b)


@jax.jit
def _embed(idx_flat, token_table, pos_table):
    mesh = plsc.VectorSubcoreMesh(core_axis_name="c", subcore_axis_name="s")
    k = pl.kernel(
        _body,
        out_type=jax.ShapeDtypeStruct((B * L, D), jnp.float32),
        mesh=mesh,
        scratch_types=[
            pltpu.VMEM((N_PER_W,), jnp.int32),
            pltpu.VMEM((L, D), jnp.float32),
            [pltpu.VMEM((C, D), jnp.float32) for _ in range(NBUF)],
            [pltpu.VMEM((C, D), jnp.float32) for _ in range(NBUF)],
            [pltpu.SemaphoreType.DMA for _ in range(NBUF)],
            [pltpu.SemaphoreType.DMA for _ in range(NBUF)],
        ],
    )
    return k(token_table, idx_flat, pos_table)


def kernel(inputs, token_table, pos_table):
    idx_flat = inputs.reshape(B * L).astype(jnp.int32)
    out = _embed(idx_flat, token_table, pos_table)
    return out.reshape(B, L, D)
